# baseline (speedup 1.0000x reference)
import functools
import math

import jax
import jax.numpy as jnp
from jax import lax
from jax.experimental import pallas as pl
from jax.experimental.pallas import tpu as pltpu
from jax.experimental.pallas import tpu_sc as plsc

F32 = jnp.float32
I32 = jnp.int32

NC = 2
NS = 16
LANES = 16
K_CH = 128
N_PASS = 5
DW = 128


def _copy_idx(dst16, src_ref, off):
    for k in range(K_CH // LANES):
        sl = pl.ds(k * LANES, LANES)
        dst16[sl] = src_ref[pl.ds(off + k * LANES, LANES)]


def _scale_chunk(rows, ewb, off):
    def scale16(k, _):
        ewv = ewb[pl.ds(off + k * LANES, LANES)]
        e0 = k * LANES
        for l in range(LANES):
            s = ewv[l]
            for j in range(DW // LANES):
                sl = pl.ds(j * LANES, LANES)
                rows[e0 + l, sl] = rows[e0 + l, sl] * s
        return 0

    lax.fori_loop(0, K_CH // LANES, scale16, 0)


def _segsum_one(tid, table, src3, dst3, ew3, out, n_pad, pt,
                srcb, dstb, ewb, sidx0, sidx1, didx0, didx1, rows0, rows1,
                acc, sem0, sem1, ssem0, ssem1):
    per_tile = n_pad // NS
    pass_pt = pt // N_PASS
    pass_ch = pass_pt // K_CH
    nsup = pass_ch // 2

    z = jnp.zeros((LANES,), F32)

    def zrow(i, _):
        r = i // (DW // LANES)
        c = i % (DW // LANES)
        rows0[r, pl.ds(c * LANES, LANES)] = z
        return 0

    lax.fori_loop(0, K_CH * (DW // LANES), zrow, 0)
    base = pl.multiple_of(tid * per_tile, 128)
    for k in range(per_tile // K_CH):
        pltpu.sync_copy(rows0, acc.at[pl.ds(base + k * K_CH, K_CH)])
    plsc.subcore_barrier()

    def drain0():
        pltpu.make_async_copy(rows0, acc.at[didx0], ssem0).wait()

    def drain1():
        pltpu.make_async_copy(rows1, acc.at[didx1], ssem1).wait()

    def do_pass(p, _):
        poff = p * pass_pt
        pltpu.sync_copy(src3.at[tid, 0, pl.ds(poff, pass_pt)], srcb)
        pltpu.sync_copy(dst3.at[tid, 0, pl.ds(poff, pass_pt)], dstb)
        pltpu.sync_copy(ew3.at[tid, 0, pl.ds(poff, pass_pt)], ewb)

        @pl.when(p > 0)
        def _():
            drain0()
            drain1()

        _copy_idx(sidx0, srcb, 0)
        pltpu.async_copy(table.at[sidx0], rows0, sem0)

        def super_chunk(i, _):
            off0 = (2 * i) * K_CH
            off1 = off0 + K_CH

            @pl.when(i > 0)
            def _():
                drain1()

            _copy_idx(sidx1, srcb, off1)
            pltpu.async_copy(table.at[sidx1], rows1, sem1)

            pltpu.make_async_copy(table.at[sidx0], rows0, sem0).wait()
            _scale_chunk(rows0, ewb, off0)
            _copy_idx(didx0, dstb, off0)
            pltpu.async_copy(rows0, acc.at[didx0], ssem0, add=True)

            pltpu.make_async_copy(table.at[sidx1], rows1, sem1).wait()
            _scale_chunk(rows1, ewb, off1)
            _copy_idx(didx1, dstb, off1)
            pltpu.async_copy(rows1, acc.at[didx1], ssem1, add=True)

            @pl.when(i + 1 < nsup)
            def _():
                drain0()
                _copy_idx(sidx0, srcb, off1 + K_CH)
                pltpu.async_copy(table.at[sidx0], rows0, sem0)

            return 0

        lax.fori_loop(0, nsup, super_chunk, 0)
        return 0

    lax.fori_loop(0, N_PASS, do_pass, 0)
    drain0()
    drain1()
    plsc.subcore_barrier()
    sl = pl.ds(base, per_tile)
    pltpu.sync_copy(acc.at[sl], out.at[sl])


def _make_edge_segsum(n_pad, pt):
    def body(t1, t2, src3_1, dst3_1, ew3_1, src3_2, dst3_2, ew3_2,
             out1, out2, srcb, dstb, ewb, sidx0, sidx1, didx0, didx1,
             rows0, rows1, acc, sem0, sem1, ssem0, ssem1):
        c = lax.axis_index("c")
        tid = lax.axis_index("s")

        @pl.when(c == 0)
        def _():
            _segsum_one(tid, t1, src3_1, dst3_1, ew3_1, out1, n_pad, pt,
                        srcb, dstb, ewb, sidx0, sidx1, didx0, didx1,
                        rows0, rows1, acc, sem0, sem1, ssem0, ssem1)

        @pl.when(c == 1)
        def _():
            _segsum_one(tid, t2, src3_2, dst3_2, ew3_2, out2, n_pad, pt,
                        srcb, dstb, ewb, sidx0, sidx1, didx0, didx1,
                        rows0, rows1, acc, sem0, sem1, ssem0, ssem1)

    return pl.kernel(
        body,
        out_type=[jax.ShapeDtypeStruct((n_pad, DW), F32),
                  jax.ShapeDtypeStruct((n_pad, DW), F32)],
        mesh=plsc.VectorSubcoreMesh(core_axis_name="c", subcore_axis_name="s"),
        scratch_types=[
            pltpu.VMEM((pt // N_PASS,), I32),
            pltpu.VMEM((pt // N_PASS,), I32),
            pltpu.VMEM((pt // N_PASS,), F32),
            pltpu.VMEM((K_CH,), I32),
            pltpu.VMEM((K_CH,), I32),
            pltpu.VMEM((K_CH,), I32),
            pltpu.VMEM((K_CH,), I32),
            pltpu.VMEM((K_CH, DW), F32),
            pltpu.VMEM((K_CH, DW), F32),
            pltpu.VMEM_SHARED((n_pad, DW), F32),
            pltpu.SemaphoreType.DMA,
            pltpu.SemaphoreType.DMA,
            pltpu.SemaphoreType.DMA,
            pltpu.SemaphoreType.DMA,
        ],
    )


def _make_deg_kernel(n_pad, pt):
    rd = n_pad // DW

    def one_graph(tid, dst3, ew3, out, dstb, ewb, acc1, acc2, ridx, sacc):
        z = jnp.zeros((LANES,), F32)

        def z2(i, _):
            r = i // (DW // LANES)
            cc = i % (DW // LANES)
            acc2[r, pl.ds(cc * LANES, LANES)] = z
            return 0

        lax.fori_loop(0, rd * (DW // LANES), z2, 0)
        zrows = rd // 8

        @pl.when(tid < zrows)
        def _():
            off = pl.multiple_of(tid * 8, 8)
            pltpu.sync_copy(acc2.at[pl.ds(0, 8)], sacc.at[pl.ds(off, 8)])

        def z1(i, _):
            acc1[pl.ds(i * LANES, LANES)] = z
            return 0

        lax.fori_loop(0, (n_pad + LANES) // LANES, z1, 0)
        pltpu.sync_copy(dst3.at[tid], dstb)
        pltpu.sync_copy(ew3.at[tid], ewb)
        plsc.subcore_barrier()

        onehot0 = jnp.where(lax.iota(I32, LANES) == 0, 1.0, 0.0).astype(F32)

        def grp(g, _):
            dv = dstb[0, pl.ds(g * LANES, LANES)]
            ev = ewb[0, pl.ds(g * LANES, LANES)]
            for l in range(LANES):
                d0 = dv[l]
                evec = onehot0 * ev[l]
                w = acc1[pl.ds(d0, LANES)]
                acc1[pl.ds(d0, LANES)] = w + evec
            return 0

        lax.fori_loop(0, pt // LANES, grp, 0)

        def repack(r, _):
            for c8 in range(DW // LANES):
                acc2[r, pl.ds(c8 * LANES, LANES)] = acc1[
                    pl.ds(r * DW + c8 * LANES, LANES)]
            return 0

        lax.fori_loop(0, rd, repack, 0)
        for k in range(rd // LANES):
            ridx[pl.ds(k * LANES, LANES)] = lax.iota(I32, LANES) + k * LANES
        pltpu.sync_copy(acc2, sacc.at[ridx], add=True)
        plsc.subcore_barrier()

        @pl.when(tid < zrows)
        def _():
            off = pl.multiple_of(tid * 8, 8)
            pltpu.sync_copy(sacc.at[pl.ds(off, 8)], out.at[pl.ds(off, 8)])

    def body(dst3_1, ew3_1, dst3_2, ew3_2, out1, out2,
             dstb, ewb, acc1, acc2, ridx, sacc):
        c = lax.axis_index("c")
        tid = lax.axis_index("s")

        @pl.when(c == 0)
        def _():
            one_graph(tid, dst3_1, ew3_1, out1, dstb, ewb, acc1, acc2, ridx,
                      sacc)

        @pl.when(c == 1)
        def _():
            one_graph(tid, dst3_2, ew3_2, out2, dstb, ewb, acc1, acc2, ridx,
                      sacc)

    return pl.kernel(
        body,
        out_type=[jax.ShapeDtypeStruct((rd, DW), F32),
                  jax.ShapeDtypeStruct((rd, DW), F32)],
        mesh=plsc.VectorSubcoreMesh(core_axis_name="c", subcore_axis_name="s"),
        scratch_types=[
            pltpu.VMEM((1, pt), I32),
            pltpu.VMEM((1, pt), F32),
            pltpu.VMEM((n_pad + LANES,), F32),
            pltpu.VMEM((rd, DW), F32),
            pltpu.VMEM((rd,), I32),
            pltpu.VMEM_SHARED((rd, DW), F32),
        ],
    )


def _make_gene_kernel(n_path, ppg, d, roots, n_ids):
    per_tile = n_ids // (NC * NS)

    def body(gtab, gids, gpath, proot, out,
             rootv, gidv, paths, pidx0, pidx1, feats0, feats1, acc0, acc1,
             psem, fsem0, fsem1, osem0, osem1):
        c = lax.axis_index("c")
        s = lax.axis_index("s")
        wid = s * NC + c
        pltpu.sync_copy(proot, rootv.at[pl.ds(0, n_path)])
        goff = pl.multiple_of(wid * per_tile, 8)
        pltpu.sync_copy(gids.at[pl.ds(goff, per_tile)], gidv)
        pltpu.async_copy(gpath.at[gidv], paths, psem).wait()

        nv = d // LANES
        z = jnp.zeros((LANES,), F32)
        obase = wid * per_tile

        def stage_pidx(pidx, g):
            for h in range(ppg // LANES):
                sl = pl.ds(h * LANES, LANES)
                pidx[sl] = paths[g, sl]

        def accumulate(acc, feats, pidx):
            def zacc(i, _):
                acc[0, pl.ds(i * LANES, LANES)] = z
                return 0

            lax.fori_loop(0, (roots * d) // LANES, zacc, 0)
            for h in range(ppg // LANES):
                pv = pidx[pl.ds(h * LANES, LANES)]
                for l in range(LANES):
                    o = pv[l]
                    r = rootv[pl.ds(o, LANES)][0] * d
                    p = h * LANES + l
                    for j in range(nv):
                        a = acc[0, pl.ds(r + j * LANES, LANES)]
                        acc[0, pl.ds(r + j * LANES, LANES)] = (
                            a + feats[p, pl.ds(j * LANES, LANES)])

        nsup = per_tile // 2
        stage_pidx(pidx0, 0)
        pltpu.async_copy(gtab.at[pidx0], feats0, fsem0)

        def super_gene(i, _):
            g0 = 2 * i
            g1 = g0 + 1
            stage_pidx(pidx1, g1)
            pltpu.async_copy(gtab.at[pidx1], feats1, fsem1)

            @pl.when(i > 0)
            def _():
                pltpu.make_async_copy(acc0, out.at[obase + g0 - 2],
                                      osem0).wait()

            pltpu.make_async_copy(gtab.at[pidx0], feats0, fsem0).wait()
            accumulate(acc0, feats0, pidx0)
            pltpu.async_copy(acc0, out.at[obase + g0], osem0)

            @pl.when(i + 1 < nsup)
            def _():
                stage_pidx(pidx0, g0 + 2)
                pltpu.async_copy(gtab.at[pidx0], feats0, fsem0)

            @pl.when(i > 0)
            def _():
                pltpu.make_async_copy(acc1, out.at[obase + g1 - 2],
                                      osem1).wait()

            pltpu.make_async_copy(gtab.at[pidx1], feats1, fsem1).wait()
            accumulate(acc1, feats1, pidx1)
            pltpu.async_copy(acc1, out.at[obase + g1], osem1)
            return 0

        lax.fori_loop(0, nsup, super_gene, 0)
        pltpu.make_async_copy(acc0, out.at[obase + per_tile - 2],
                              osem0).wait()
        pltpu.make_async_copy(acc1, out.at[obase + per_tile - 1],
                              osem1).wait()

    return pl.kernel(
        body,
        out_type=jax.ShapeDtypeStruct((n_ids, 1, roots * d), F32),
        mesh=plsc.VectorSubcoreMesh(core_axis_name="c", subcore_axis_name="s"),
        scratch_types=[
            pltpu.VMEM((n_path + LANES,), I32),
            pltpu.VMEM((per_tile,), I32),
            pltpu.VMEM((per_tile, DW), I32),
            pltpu.VMEM((ppg,), I32),
            pltpu.VMEM((ppg,), I32),
            pltpu.VMEM((ppg, DW), F32),
            pltpu.VMEM((ppg, DW), F32),
            pltpu.VMEM((1, roots * d), F32),
            pltpu.VMEM((1, roots * d), F32),
            pltpu.SemaphoreType.DMA,
            pltpu.SemaphoreType.DMA,
            pltpu.SemaphoreType.DMA,
            pltpu.SemaphoreType.DMA,
            pltpu.SemaphoreType.DMA,
        ],
    )


_DOT = functools.partial(jnp.dot, preferred_element_type=F32,
                         precision=lax.Precision.HIGHEST)


def _tca_body(x_ref, w_ref, deg_ref, oa_ref, ob_ref):
    dinv = lax.rsqrt(deg_ref[...] + 1.0)
    h = _DOT(x_ref[...], w_ref[...]) * dinv
    oa_ref[...] = h[:, :h.shape[1] // 2]
    ob_ref[...] = h[:, h.shape[1] // 2:]


def _make_tca(n, d_in, h1, rblk):
    return pl.pallas_call(
        _tca_body,
        grid=(n // rblk,),
        in_specs=[
            pl.BlockSpec((rblk, d_in), lambda i: (i, 0)),
            pl.BlockSpec((d_in, h1), lambda i: (0, 0)),
            pl.BlockSpec((rblk, 1), lambda i: (i, 0)),
        ],
        out_specs=[
            pl.BlockSpec((rblk, h1 // 2), lambda i: (i, 0)),
            pl.BlockSpec((rblk, h1 // 2), lambda i: (i, 0)),
        ],
        out_shape=[jax.ShapeDtypeStruct((n, h1 // 2), F32),
                   jax.ShapeDtypeStruct((n, h1 // 2), F32)],
    )


def _tcb_body(sa_ref, sb_ref, ha_ref, hb_ref, deg_ref, b1_ref, w2_ref, o_ref):
    dinv = lax.rsqrt(deg_ref[...] + 1.0)
    xa = (sa_ref[...] + ha_ref[...]) * dinv
    xb = (sb_ref[...] + hb_ref[...]) * dinv
    xr = jnp.maximum(jnp.concatenate([xa, xb], axis=1) + b1_ref[...], 0.0)
    h2 = _DOT(xr, w2_ref[...]) * dinv
    o_ref[...] = jnp.concatenate(
        [h2, jnp.zeros((h2.shape[0], DW - h2.shape[1]), F32)], axis=1)


def _make_tcb(n, h1, d_out, rblk):
    hh = h1 // 2
    return pl.pallas_call(
        _tcb_body,
        grid=(n // rblk,),
        in_specs=[
            pl.BlockSpec((rblk, hh), lambda i: (i, 0)),
            pl.BlockSpec((rblk, hh), lambda i: (i, 0)),
            pl.BlockSpec((rblk, hh), lambda i: (i, 0)),
            pl.BlockSpec((rblk, hh), lambda i: (i, 0)),
            pl.BlockSpec((rblk, 1), lambda i: (i, 0)),
            pl.BlockSpec((1, h1), lambda i: (0, 0)),
            pl.BlockSpec((h1, d_out), lambda i: (0, 0)),
        ],
        out_specs=pl.BlockSpec((rblk, DW), lambda i: (i, 0)),
        out_shape=jax.ShapeDtypeStruct((n, DW), F32),
    )


def _tcc_body(s1_ref, h1_ref, d1_ref, b1_ref, s2_ref, h2_ref, d2_ref, b2_ref,
              o_ref):
    d_out = b1_ref.shape[1]
    dinv1 = lax.rsqrt(d1_ref[...] + 1.0)
    dinv2 = lax.rsqrt(d2_ref[...] + 1.0)
    g1 = (s1_ref[...][:, :d_out] + h1_ref[...][:, :d_out]) * dinv1 + b1_ref[...]
    g2 = (s2_ref[...][:, :d_out] + h2_ref[...][:, :d_out]) * dinv2 + b2_ref[...]
    g = g1 + g2
    o_ref[...] = jnp.concatenate(
        [g, jnp.zeros((g.shape[0], DW - d_out), F32)], axis=1)


def _make_tcc(n, d_out, rblk):
    return pl.pallas_call(
        _tcc_body,
        grid=(n // rblk,),
        in_specs=[
            pl.BlockSpec((rblk, DW), lambda i: (i, 0)),
            pl.BlockSpec((rblk, DW), lambda i: (i, 0)),
            pl.BlockSpec((rblk, 1), lambda i: (i, 0)),
            pl.BlockSpec((1, d_out), lambda i: (0, 0)),
            pl.BlockSpec((rblk, DW), lambda i: (i, 0)),
            pl.BlockSpec((rblk, DW), lambda i: (i, 0)),
            pl.BlockSpec((rblk, 1), lambda i: (i, 0)),
            pl.BlockSpec((1, d_out), lambda i: (0, 0)),
        ],
        out_specs=pl.BlockSpec((rblk, DW), lambda i: (i, 0)),
        out_shape=jax.ShapeDtypeStruct((n, DW), F32),
    )


_BN_C = float(1.0 / math.sqrt(1.0 + 1e-5))


def _tail1_body(x_ref, w1_ref, b1_ref, g1_ref, bb1_ref, w2_ref, b2_ref,
                g2_ref, bb2_ref, o_ref):
    z = _DOT(x_ref[...], w1_ref[...]) + b1_ref[...]
    z = jnp.maximum(z * _BN_C * g1_ref[...] + bb1_ref[...], 0.0)
    z2 = _DOT(z, w2_ref[...]) + b2_ref[...]
    o_ref[...] = z2 * _BN_C * g2_ref[...] + bb2_ref[...]


def _make_tail1(n, din, h, dout, rblk):
    return pl.pallas_call(
        _tail1_body,
        grid=(n // rblk,),
        in_specs=[
            pl.BlockSpec((rblk, din), lambda i: (i, 0)),
            pl.BlockSpec((din, h), lambda i: (0, 0)),
            pl.BlockSpec((1, h), lambda i: (0, 0)),
            pl.BlockSpec((1, h), lambda i: (0, 0)),
            pl.BlockSpec((1, h), lambda i: (0, 0)),
            pl.BlockSpec((h, dout), lambda i: (0, 0)),
            pl.BlockSpec((1, dout), lambda i: (0, 0)),
            pl.BlockSpec((1, dout), lambda i: (0, 0)),
            pl.BlockSpec((1, dout), lambda i: (0, 0)),
        ],
        out_specs=pl.BlockSpec((rblk, dout), lambda i: (i, 0)),
        out_shape=jax.ShapeDtypeStruct((n, dout), F32),
    )


def _tail2_body(fh_ref, ft_ref, wf_ref, bf_ref, w1_ref, b1_ref, w2_ref,
                b2_ref, o_ref):
    fuse = jnp.concatenate([fh_ref[...], ft_ref[...]], axis=1)
    z1 = jnp.maximum(_DOT(fuse, wf_ref[...]) + bf_ref[...], 0.0)
    ff = _DOT(z1, w1_ref[...]) + b1_ref[...]
    z2 = jnp.maximum(ff, 0.0)
    o_ref[...] = jnp.sum(z2 * w2_ref[...], axis=1, keepdims=True) + b2_ref[...]


def _make_tail2(b, gdim, h1, h2, rblk):
    nb = b // rblk
    return pl.pallas_call(
        _tail2_body,
        grid=(nb,),
        in_specs=[
            pl.BlockSpec((rblk, gdim), lambda i: (i, 0)),
            pl.BlockSpec((rblk, gdim), lambda i: (i + nb, 0)),
            pl.BlockSpec((2 * gdim, h1), lambda i: (0, 0)),
            pl.BlockSpec((1, h1), lambda i: (0, 0)),
            pl.BlockSpec((h1, h2), lambda i: (0, 0)),
            pl.BlockSpec((1, h2), lambda i: (0, 0)),
            pl.BlockSpec((1, h2), lambda i: (0, 0)),
            pl.BlockSpec((1, 1), lambda i: (0, 0)),
        ],
        out_specs=pl.BlockSpec((rblk, 1), lambda i: (i, 0)),
        out_shape=jax.ShapeDtypeStruct((b, 1), F32),
    )


def _edges3(edge_index, edge_weight):
    src = edge_index[0].astype(I32)
    dst = edge_index[1].astype(I32)
    w = edge_weight.astype(F32)
    e = src.shape[0]
    gran = NS * K_CH * N_PASS
    e_pad = ((e + gran - 1) // gran) * gran
    pad = e_pad - e
    if pad:
        src = jnp.concatenate([src, jnp.zeros((pad,), I32)])
        dst = jnp.concatenate([dst, jnp.zeros((pad,), I32)])
        w = jnp.concatenate([w, jnp.zeros((pad,), F32)])
    pt = e_pad // NS
    shp = (NS, 1, pt)
    return src.reshape(shp), dst.reshape(shp), w.reshape(shp), pt


def kernel(head_ids, tail_ids, x1, x2, edge_index1, edge_index2, edge_weight1,
           edge_weight2, gene_pathways, pathway_root, g1_W1, g1_b1, g1_W2,
           g1_b2, g2_W1, g2_b1, g2_W2, g2_b2, pl1_W, pl1_b, pl2_W, pl2_b,
           bn1_g, bn1_b, bn2_g, bn2_b, fc_W, fc_b, fc1_W, fc1_b, fc2_W,
           fc2_b):
    n = x1.shape[0]
    d_in = x1.shape[1]
    h1 = g1_W1.shape[1]
    d_out = g1_W2.shape[1]
    b = head_ids.shape[0]
    ppg = gene_pathways.shape[1]
    roots = pl1_W.shape[0] // d_out
    gdim = pl2_W.shape[1]

    src3_1, dst3_1, ew3_1, pt = _edges3(edge_index1, edge_weight1)
    src3_2, dst3_2, ew3_2, _ = _edges3(edge_index2, edge_weight2)

    n_pad = ((n + (NS * K_CH) - 1) // (NS * K_CH)) * (NS * K_CH)

    segsum = _make_edge_segsum(n_pad, pt)

    deg_k = _make_deg_kernel(n_pad, pt)
    D1, D2 = deg_k(dst3_1, ew3_1, dst3_2, ew3_2)
    deg1 = D1.reshape(n_pad)[:n, None]
    deg2 = D2.reshape(n_pad)[:n, None]

    rblk = 1000
    tca = _make_tca(n, d_in, h1, rblk)
    hA1, hB1 = tca(x1, g1_W1, deg1)
    hA2, hB2 = tca(x2, g2_W1, deg2)

    SA1, SB1 = segsum(hA1, hB1, src3_1, dst3_1, ew3_1,
                      src3_1, dst3_1, ew3_1)
    SA2, SB2 = segsum(hA2, hB2, src3_2, dst3_2, ew3_2,
                      src3_2, dst3_2, ew3_2)

    tcb = _make_tcb(n, h1, d_out, rblk)
    h2p1 = tcb(SA1[:n], SB1[:n], hA1, hB1, deg1, g1_b1.reshape(1, -1), g1_W2)
    h2p2 = tcb(SA2[:n], SB2[:n], hA2, hB2, deg2, g2_b1.reshape(1, -1), g2_W2)

    S21, S22 = segsum(h2p1, h2p2, src3_1, dst3_1, ew3_1,
                      src3_2, dst3_2, ew3_2)

    tcc = _make_tcc(n, d_out, rblk)
    gsum = tcc(S21[:n], h2p1, deg1, g1_b2.reshape(1, -1),
               S22[:n], h2p2, deg2, g2_b2.reshape(1, -1))

    gids = jnp.concatenate([head_ids, tail_ids]).astype(I32)
    gp_pad = jnp.concatenate(
        [gene_pathways.astype(I32),
         jnp.zeros((gene_pathways.shape[0], DW - ppg), I32)], axis=1)
    gene_k = _make_gene_kernel(n, ppg, d_out, roots, 2 * b)
    HT = gene_k(gsum, gids, gp_pad, pathway_root.astype(I32))
    HTf = HT.reshape(2 * b, roots * d_out)

    tail1 = _make_tail1(2 * b, roots * d_out, pl1_W.shape[1], gdim, 256)
    F = tail1(HTf, pl1_W, pl1_b.reshape(1, -1), bn1_g.reshape(1, -1),
              bn1_b.reshape(1, -1), pl2_W, pl2_b.reshape(1, -1),
              bn2_g.reshape(1, -1), bn2_b.reshape(1, -1))

    tail2 = _make_tail2(b, gdim, fc_W.shape[1], fc1_W.shape[1], 256)
    out = tail2(F, F, fc_W, fc_b.reshape(1, -1), fc1_W, fc1_b.reshape(1, -1),
                fc2_W.reshape(1, -1), fc2_b.reshape(1, 1))
    return out


# speedup vs baseline: 1.0321x; 1.0321x over previous
import functools
import math

import jax
import jax.numpy as jnp
from jax import lax
from jax.experimental import pallas as pl
from jax.experimental.pallas import tpu as pltpu
from jax.experimental.pallas import tpu_sc as plsc

F32 = jnp.float32
I32 = jnp.int32

NC = 2
NS = 16
LANES = 16
K_CH = 128
N_PASS = 5
DW = 128


def _copy_idx(dst16, src_ref, off):
    for k in range(K_CH // LANES):
        sl = pl.ds(k * LANES, LANES)
        dst16[sl] = src_ref[pl.ds(off + k * LANES, LANES)]


def _scale_chunk(rows, ewb, off):
    def scale16(k, _):
        ewv = ewb[pl.ds(off + k * LANES, LANES)]
        e0 = k * LANES
        for l in range(LANES):
            s = ewv[l]
            for j in range(DW // LANES):
                sl = pl.ds(j * LANES, LANES)
                rows[e0 + l, sl] = rows[e0 + l, sl] * s
        return 0

    lax.fori_loop(0, K_CH // LANES, scale16, 0)


def _segsum_one(tid, table, src3, dst3, ew3, out, n_pad, pt,
                srcb, dstb, ewb, sidx0, sidx1, didx0, didx1, rows0, rows1,
                acc, sem0, sem1, ssem0, ssem1):
    per_tile = n_pad // NS
    pass_pt = pt // N_PASS
    pass_ch = pass_pt // K_CH
    nsup = pass_ch // 2

    z = jnp.zeros((LANES,), F32)

    def zrow(i, _):
        r = i // (DW // LANES)
        c = i % (DW // LANES)
        rows0[r, pl.ds(c * LANES, LANES)] = z
        return 0

    lax.fori_loop(0, K_CH * (DW // LANES), zrow, 0)
    base = pl.multiple_of(tid * per_tile, 128)
    for k in range(per_tile // K_CH):
        pltpu.sync_copy(rows0, acc.at[pl.ds(base + k * K_CH, K_CH)])
    plsc.subcore_barrier()

    def drain0():
        pltpu.make_async_copy(rows0, acc.at[didx0], ssem0).wait()

    def drain1():
        pltpu.make_async_copy(rows1, acc.at[didx1], ssem1).wait()

    def do_pass(p, _):
        poff = p * pass_pt
        pltpu.sync_copy(src3.at[tid, 0, pl.ds(poff, pass_pt)], srcb)
        pltpu.sync_copy(dst3.at[tid, 0, pl.ds(poff, pass_pt)], dstb)
        pltpu.sync_copy(ew3.at[tid, 0, pl.ds(poff, pass_pt)], ewb)

        @pl.when(p > 0)
        def _():
            drain0()
            drain1()

        _copy_idx(sidx0, srcb, 0)
        pltpu.async_copy(table.at[sidx0], rows0, sem0)

        def super_chunk(i, _):
            off0 = (2 * i) * K_CH
            off1 = off0 + K_CH

            @pl.when(i > 0)
            def _():
                drain1()

            _copy_idx(sidx1, srcb, off1)
            pltpu.async_copy(table.at[sidx1], rows1, sem1)

            pltpu.make_async_copy(table.at[sidx0], rows0, sem0).wait()
            _scale_chunk(rows0, ewb, off0)
            _copy_idx(didx0, dstb, off0)
            pltpu.async_copy(rows0, acc.at[didx0], ssem0, add=True)

            pltpu.make_async_copy(table.at[sidx1], rows1, sem1).wait()
            _scale_chunk(rows1, ewb, off1)
            _copy_idx(didx1, dstb, off1)
            pltpu.async_copy(rows1, acc.at[didx1], ssem1, add=True)

            @pl.when(i + 1 < nsup)
            def _():
                drain0()
                _copy_idx(sidx0, srcb, off1 + K_CH)
                pltpu.async_copy(table.at[sidx0], rows0, sem0)

            return 0

        lax.fori_loop(0, nsup, super_chunk, 0)
        return 0

    lax.fori_loop(0, N_PASS, do_pass, 0)
    drain0()
    drain1()
    plsc.subcore_barrier()
    sl = pl.ds(base, per_tile)
    pltpu.sync_copy(acc.at[sl], out.at[sl])


def _make_edge_segsum(n_pad, pt):
    def body(t1, t2, src3_1, dst3_1, ew3_1, src3_2, dst3_2, ew3_2,
             out1, out2, srcb, dstb, ewb, sidx0, sidx1, didx0, didx1,
             rows0, rows1, acc, sem0, sem1, ssem0, ssem1):
        c = lax.axis_index("c")
        tid = lax.axis_index("s")

        @pl.when(c == 0)
        def _():
            _segsum_one(tid, t1, src3_1, dst3_1, ew3_1, out1, n_pad, pt,
                        srcb, dstb, ewb, sidx0, sidx1, didx0, didx1,
                        rows0, rows1, acc, sem0, sem1, ssem0, ssem1)

        @pl.when(c == 1)
        def _():
            _segsum_one(tid, t2, src3_2, dst3_2, ew3_2, out2, n_pad, pt,
                        srcb, dstb, ewb, sidx0, sidx1, didx0, didx1,
                        rows0, rows1, acc, sem0, sem1, ssem0, ssem1)

    return pl.kernel(
        body,
        out_type=[jax.ShapeDtypeStruct((n_pad, DW), F32),
                  jax.ShapeDtypeStruct((n_pad, DW), F32)],
        mesh=plsc.VectorSubcoreMesh(core_axis_name="c", subcore_axis_name="s"),
        scratch_types=[
            pltpu.VMEM((pt // N_PASS,), I32),
            pltpu.VMEM((pt // N_PASS,), I32),
            pltpu.VMEM((pt // N_PASS,), F32),
            pltpu.VMEM((K_CH,), I32),
            pltpu.VMEM((K_CH,), I32),
            pltpu.VMEM((K_CH,), I32),
            pltpu.VMEM((K_CH,), I32),
            pltpu.VMEM((K_CH, DW), F32),
            pltpu.VMEM((K_CH, DW), F32),
            pltpu.VMEM_SHARED((n_pad, DW), F32),
            pltpu.SemaphoreType.DMA,
            pltpu.SemaphoreType.DMA,
            pltpu.SemaphoreType.DMA,
            pltpu.SemaphoreType.DMA,
        ],
    )


def _make_deg_kernel(n_pad, pt):
    rd = n_pad // DW

    def one_graph(tid, dst3, ew3, out, dstb, ewb, acc1, acc2, ridx, sacc):
        z = jnp.zeros((LANES,), F32)

        def z2(i, _):
            r = i // (DW // LANES)
            cc = i % (DW // LANES)
            acc2[r, pl.ds(cc * LANES, LANES)] = z
            return 0

        lax.fori_loop(0, rd * (DW // LANES), z2, 0)
        zrows = rd // 8

        @pl.when(tid < zrows)
        def _():
            off = pl.multiple_of(tid * 8, 8)
            pltpu.sync_copy(acc2.at[pl.ds(0, 8)], sacc.at[pl.ds(off, 8)])

        def z1(i, _):
            acc1[pl.ds(i * LANES, LANES)] = z
            return 0

        lax.fori_loop(0, (n_pad + LANES) // LANES, z1, 0)
        pltpu.sync_copy(dst3.at[tid], dstb)
        pltpu.sync_copy(ew3.at[tid], ewb)
        plsc.subcore_barrier()

        onehot0 = jnp.where(lax.iota(I32, LANES) == 0, 1.0, 0.0).astype(F32)

        def grp(g, _):
            dv = dstb[0, pl.ds(g * LANES, LANES)]
            ev = ewb[0, pl.ds(g * LANES, LANES)]
            for l in range(LANES):
                d0 = dv[l]
                evec = onehot0 * ev[l]
                w = acc1[pl.ds(d0, LANES)]
                acc1[pl.ds(d0, LANES)] = w + evec
            return 0

        lax.fori_loop(0, pt // LANES, grp, 0)

        def repack(r, _):
            for c8 in range(DW // LANES):
                acc2[r, pl.ds(c8 * LANES, LANES)] = acc1[
                    pl.ds(r * DW + c8 * LANES, LANES)]
            return 0

        lax.fori_loop(0, rd, repack, 0)
        for k in range(rd // LANES):
            ridx[pl.ds(k * LANES, LANES)] = lax.iota(I32, LANES) + k * LANES
        pltpu.sync_copy(acc2, sacc.at[ridx], add=True)
        plsc.subcore_barrier()

        @pl.when(tid < zrows)
        def _():
            off = pl.multiple_of(tid * 8, 8)
            pltpu.sync_copy(sacc.at[pl.ds(off, 8)], out.at[pl.ds(off, 8)])

    def body(dst3_1, ew3_1, dst3_2, ew3_2, out1, out2,
             dstb, ewb, acc1, acc2, ridx, sacc):
        c = lax.axis_index("c")
        tid = lax.axis_index("s")

        @pl.when(c == 0)
        def _():
            one_graph(tid, dst3_1, ew3_1, out1, dstb, ewb, acc1, acc2, ridx,
                      sacc)

        @pl.when(c == 1)
        def _():
            one_graph(tid, dst3_2, ew3_2, out2, dstb, ewb, acc1, acc2, ridx,
                      sacc)

    return pl.kernel(
        body,
        out_type=[jax.ShapeDtypeStruct((rd, DW), F32),
                  jax.ShapeDtypeStruct((rd, DW), F32)],
        mesh=plsc.VectorSubcoreMesh(core_axis_name="c", subcore_axis_name="s"),
        scratch_types=[
            pltpu.VMEM((1, pt), I32),
            pltpu.VMEM((1, pt), F32),
            pltpu.VMEM((n_pad + LANES,), F32),
            pltpu.VMEM((rd, DW), F32),
            pltpu.VMEM((rd,), I32),
            pltpu.VMEM_SHARED((rd, DW), F32),
        ],
    )


def _make_gene_kernel(n_path, ppg, d, roots, n_ids):
    per_tile = n_ids // (NC * NS)

    def body(gtab, gids, gpath, proot, out,
             rootv, gidv, paths, pidx0, pidx1, feats0, feats1, acc0, acc1,
             psem, fsem0, fsem1, osem0, osem1):
        c = lax.axis_index("c")
        s = lax.axis_index("s")
        wid = s * NC + c
        pltpu.sync_copy(proot, rootv.at[pl.ds(0, n_path)])
        goff = pl.multiple_of(wid * per_tile, 8)
        pltpu.sync_copy(gids.at[pl.ds(goff, per_tile)], gidv)
        pltpu.async_copy(gpath.at[gidv], paths, psem).wait()

        nv = d // LANES
        z = jnp.zeros((LANES,), F32)
        obase = wid * per_tile

        def stage_pidx(pidx, g):
            for h in range(ppg // LANES):
                sl = pl.ds(h * LANES, LANES)
                pidx[sl] = paths[g, sl]

        def accumulate(acc, feats, pidx):
            def zacc(i, _):
                acc[0, pl.ds(i * LANES, LANES)] = z
                return 0

            lax.fori_loop(0, (roots * d) // LANES, zacc, 0)
            for h in range(ppg // LANES):
                pv = pidx[pl.ds(h * LANES, LANES)]
                for l in range(LANES):
                    o = pv[l]
                    r = rootv[pl.ds(o, LANES)][0] * d
                    p = h * LANES + l
                    for j in range(nv):
                        a = acc[0, pl.ds(r + j * LANES, LANES)]
                        acc[0, pl.ds(r + j * LANES, LANES)] = (
                            a + feats[p, pl.ds(j * LANES, LANES)])

        nsup = per_tile // 2
        stage_pidx(pidx0, 0)
        pltpu.async_copy(gtab.at[pidx0], feats0, fsem0)

        def super_gene(i, _):
            g0 = 2 * i
            g1 = g0 + 1
            stage_pidx(pidx1, g1)
            pltpu.async_copy(gtab.at[pidx1], feats1, fsem1)

            @pl.when(i > 0)
            def _():
                pltpu.make_async_copy(acc0, out.at[obase + g0 - 2],
                                      osem0).wait()

            pltpu.make_async_copy(gtab.at[pidx0], feats0, fsem0).wait()
            accumulate(acc0, feats0, pidx0)
            pltpu.async_copy(acc0, out.at[obase + g0], osem0)

            @pl.when(i + 1 < nsup)
            def _():
                stage_pidx(pidx0, g0 + 2)
                pltpu.async_copy(gtab.at[pidx0], feats0, fsem0)

            @pl.when(i > 0)
            def _():
                pltpu.make_async_copy(acc1, out.at[obase + g1 - 2],
                                      osem1).wait()

            pltpu.make_async_copy(gtab.at[pidx1], feats1, fsem1).wait()
            accumulate(acc1, feats1, pidx1)
            pltpu.async_copy(acc1, out.at[obase + g1], osem1)
            return 0

        lax.fori_loop(0, nsup, super_gene, 0)
        pltpu.make_async_copy(acc0, out.at[obase + per_tile - 2],
                              osem0).wait()
        pltpu.make_async_copy(acc1, out.at[obase + per_tile - 1],
                              osem1).wait()

    return pl.kernel(
        body,
        out_type=jax.ShapeDtypeStruct((n_ids, 1, roots * d), F32),
        mesh=plsc.VectorSubcoreMesh(core_axis_name="c", subcore_axis_name="s"),
        scratch_types=[
            pltpu.VMEM((n_path + LANES,), I32),
            pltpu.VMEM((per_tile,), I32),
            pltpu.VMEM((per_tile, DW), I32),
            pltpu.VMEM((ppg,), I32),
            pltpu.VMEM((ppg,), I32),
            pltpu.VMEM((ppg, DW), F32),
            pltpu.VMEM((ppg, DW), F32),
            pltpu.VMEM((1, roots * d), F32),
            pltpu.VMEM((1, roots * d), F32),
            pltpu.SemaphoreType.DMA,
            pltpu.SemaphoreType.DMA,
            pltpu.SemaphoreType.DMA,
            pltpu.SemaphoreType.DMA,
            pltpu.SemaphoreType.DMA,
        ],
    )


_DOT = functools.partial(jnp.dot, preferred_element_type=F32)


def _tca_body(x_ref, w_ref, deg_ref, oa_ref, ob_ref):
    dinv = lax.rsqrt(deg_ref[...] + 1.0)
    h = _DOT(x_ref[...], w_ref[...]) * dinv
    oa_ref[...] = h[:, :h.shape[1] // 2]
    ob_ref[...] = h[:, h.shape[1] // 2:]


def _make_tca(n, d_in, h1, rblk):
    return pl.pallas_call(
        _tca_body,
        grid=(n // rblk,),
        in_specs=[
            pl.BlockSpec((rblk, d_in), lambda i: (i, 0)),
            pl.BlockSpec((d_in, h1), lambda i: (0, 0)),
            pl.BlockSpec((rblk, 1), lambda i: (i, 0)),
        ],
        out_specs=[
            pl.BlockSpec((rblk, h1 // 2), lambda i: (i, 0)),
            pl.BlockSpec((rblk, h1 // 2), lambda i: (i, 0)),
        ],
        out_shape=[jax.ShapeDtypeStruct((n, h1 // 2), F32),
                   jax.ShapeDtypeStruct((n, h1 // 2), F32)],
    )


def _tcb_body(sa_ref, sb_ref, ha_ref, hb_ref, deg_ref, b1_ref, w2_ref, o_ref):
    dinv = lax.rsqrt(deg_ref[...] + 1.0)
    xa = (sa_ref[...] + ha_ref[...]) * dinv
    xb = (sb_ref[...] + hb_ref[...]) * dinv
    xr = jnp.maximum(jnp.concatenate([xa, xb], axis=1) + b1_ref[...], 0.0)
    h2 = _DOT(xr, w2_ref[...]) * dinv
    o_ref[...] = jnp.concatenate(
        [h2, jnp.zeros((h2.shape[0], DW - h2.shape[1]), F32)], axis=1)


def _make_tcb(n, h1, d_out, rblk):
    hh = h1 // 2
    return pl.pallas_call(
        _tcb_body,
        grid=(n // rblk,),
        in_specs=[
            pl.BlockSpec((rblk, hh), lambda i: (i, 0)),
            pl.BlockSpec((rblk, hh), lambda i: (i, 0)),
            pl.BlockSpec((rblk, hh), lambda i: (i, 0)),
            pl.BlockSpec((rblk, hh), lambda i: (i, 0)),
            pl.BlockSpec((rblk, 1), lambda i: (i, 0)),
            pl.BlockSpec((1, h1), lambda i: (0, 0)),
            pl.BlockSpec((h1, d_out), lambda i: (0, 0)),
        ],
        out_specs=pl.BlockSpec((rblk, DW), lambda i: (i, 0)),
        out_shape=jax.ShapeDtypeStruct((n, DW), F32),
    )


def _tcc_body(s1_ref, h1_ref, d1_ref, b1_ref, s2_ref, h2_ref, d2_ref, b2_ref,
              o_ref):
    d_out = b1_ref.shape[1]
    dinv1 = lax.rsqrt(d1_ref[...] + 1.0)
    dinv2 = lax.rsqrt(d2_ref[...] + 1.0)
    g1 = (s1_ref[...][:, :d_out] + h1_ref[...][:, :d_out]) * dinv1 + b1_ref[...]
    g2 = (s2_ref[...][:, :d_out] + h2_ref[...][:, :d_out]) * dinv2 + b2_ref[...]
    g = g1 + g2
    o_ref[...] = jnp.concatenate(
        [g, jnp.zeros((g.shape[0], DW - d_out), F32)], axis=1)


def _make_tcc(n, d_out, rblk):
    return pl.pallas_call(
        _tcc_body,
        grid=(n // rblk,),
        in_specs=[
            pl.BlockSpec((rblk, DW), lambda i: (i, 0)),
            pl.BlockSpec((rblk, DW), lambda i: (i, 0)),
            pl.BlockSpec((rblk, 1), lambda i: (i, 0)),
            pl.BlockSpec((1, d_out), lambda i: (0, 0)),
            pl.BlockSpec((rblk, DW), lambda i: (i, 0)),
            pl.BlockSpec((rblk, DW), lambda i: (i, 0)),
            pl.BlockSpec((rblk, 1), lambda i: (i, 0)),
            pl.BlockSpec((1, d_out), lambda i: (0, 0)),
        ],
        out_specs=pl.BlockSpec((rblk, DW), lambda i: (i, 0)),
        out_shape=jax.ShapeDtypeStruct((n, DW), F32),
    )


_BN_C = float(1.0 / math.sqrt(1.0 + 1e-5))


def _tail1_body(x_ref, w1_ref, b1_ref, g1_ref, bb1_ref, w2_ref, b2_ref,
                g2_ref, bb2_ref, o_ref):
    z = _DOT(x_ref[...], w1_ref[...]) + b1_ref[...]
    z = jnp.maximum(z * _BN_C * g1_ref[...] + bb1_ref[...], 0.0)
    z2 = _DOT(z, w2_ref[...]) + b2_ref[...]
    o_ref[...] = z2 * _BN_C * g2_ref[...] + bb2_ref[...]


def _make_tail1(n, din, h, dout, rblk):
    return pl.pallas_call(
        _tail1_body,
        grid=(n // rblk,),
        in_specs=[
            pl.BlockSpec((rblk, din), lambda i: (i, 0)),
            pl.BlockSpec((din, h), lambda i: (0, 0)),
            pl.BlockSpec((1, h), lambda i: (0, 0)),
            pl.BlockSpec((1, h), lambda i: (0, 0)),
            pl.BlockSpec((1, h), lambda i: (0, 0)),
            pl.BlockSpec((h, dout), lambda i: (0, 0)),
            pl.BlockSpec((1, dout), lambda i: (0, 0)),
            pl.BlockSpec((1, dout), lambda i: (0, 0)),
            pl.BlockSpec((1, dout), lambda i: (0, 0)),
        ],
        out_specs=pl.BlockSpec((rblk, dout), lambda i: (i, 0)),
        out_shape=jax.ShapeDtypeStruct((n, dout), F32),
    )


def _tail2_body(fh_ref, ft_ref, wf_ref, bf_ref, w1_ref, b1_ref, w2_ref,
                b2_ref, o_ref):
    fuse = jnp.concatenate([fh_ref[...], ft_ref[...]], axis=1)
    z1 = jnp.maximum(_DOT(fuse, wf_ref[...]) + bf_ref[...], 0.0)
    ff = _DOT(z1, w1_ref[...]) + b1_ref[...]
    z2 = jnp.maximum(ff, 0.0)
    o_ref[...] = jnp.sum(z2 * w2_ref[...], axis=1, keepdims=True) + b2_ref[...]


def _make_tail2(b, gdim, h1, h2, rblk):
    nb = b // rblk
    return pl.pallas_call(
        _tail2_body,
        grid=(nb,),
        in_specs=[
            pl.BlockSpec((rblk, gdim), lambda i: (i, 0)),
            pl.BlockSpec((rblk, gdim), lambda i: (i + nb, 0)),
            pl.BlockSpec((2 * gdim, h1), lambda i: (0, 0)),
            pl.BlockSpec((1, h1), lambda i: (0, 0)),
            pl.BlockSpec((h1, h2), lambda i: (0, 0)),
            pl.BlockSpec((1, h2), lambda i: (0, 0)),
            pl.BlockSpec((1, h2), lambda i: (0, 0)),
            pl.BlockSpec((1, 1), lambda i: (0, 0)),
        ],
        out_specs=pl.BlockSpec((rblk, 1), lambda i: (i, 0)),
        out_shape=jax.ShapeDtypeStruct((b, 1), F32),
    )


def _edges3(edge_index, edge_weight):
    src = edge_index[0].astype(I32)
    dst = edge_index[1].astype(I32)
    w = edge_weight.astype(F32)
    e = src.shape[0]
    gran = NS * K_CH * N_PASS
    e_pad = ((e + gran - 1) // gran) * gran
    pad = e_pad - e
    if pad:
        src = jnp.concatenate([src, jnp.zeros((pad,), I32)])
        dst = jnp.concatenate([dst, jnp.zeros((pad,), I32)])
        w = jnp.concatenate([w, jnp.zeros((pad,), F32)])
    pt = e_pad // NS
    shp = (NS, 1, pt)
    return src.reshape(shp), dst.reshape(shp), w.reshape(shp), pt


def kernel(head_ids, tail_ids, x1, x2, edge_index1, edge_index2, edge_weight1,
           edge_weight2, gene_pathways, pathway_root, g1_W1, g1_b1, g1_W2,
           g1_b2, g2_W1, g2_b1, g2_W2, g2_b2, pl1_W, pl1_b, pl2_W, pl2_b,
           bn1_g, bn1_b, bn2_g, bn2_b, fc_W, fc_b, fc1_W, fc1_b, fc2_W,
           fc2_b):
    n = x1.shape[0]
    d_in = x1.shape[1]
    h1 = g1_W1.shape[1]
    d_out = g1_W2.shape[1]
    b = head_ids.shape[0]
    ppg = gene_pathways.shape[1]
    roots = pl1_W.shape[0] // d_out
    gdim = pl2_W.shape[1]

    src3_1, dst3_1, ew3_1, pt = _edges3(edge_index1, edge_weight1)
    src3_2, dst3_2, ew3_2, _ = _edges3(edge_index2, edge_weight2)

    n_pad = ((n + (NS * K_CH) - 1) // (NS * K_CH)) * (NS * K_CH)

    segsum = _make_edge_segsum(n_pad, pt)

    deg_k = _make_deg_kernel(n_pad, pt)
    D1, D2 = deg_k(dst3_1, ew3_1, dst3_2, ew3_2)
    deg1 = D1.reshape(n_pad)[:n, None]
    deg2 = D2.reshape(n_pad)[:n, None]

    rblk = 1000
    tca = _make_tca(n, d_in, h1, rblk)
    hA1, hB1 = tca(x1, g1_W1, deg1)
    hA2, hB2 = tca(x2, g2_W1, deg2)

    SA1, SB1 = segsum(hA1, hB1, src3_1, dst3_1, ew3_1,
                      src3_1, dst3_1, ew3_1)
    SA2, SB2 = segsum(hA2, hB2, src3_2, dst3_2, ew3_2,
                      src3_2, dst3_2, ew3_2)

    tcb = _make_tcb(n, h1, d_out, rblk)
    h2p1 = tcb(SA1[:n], SB1[:n], hA1, hB1, deg1, g1_b1.reshape(1, -1), g1_W2)
    h2p2 = tcb(SA2[:n], SB2[:n], hA2, hB2, deg2, g2_b1.reshape(1, -1), g2_W2)

    S21, S22 = segsum(h2p1, h2p2, src3_1, dst3_1, ew3_1,
                      src3_2, dst3_2, ew3_2)

    tcc = _make_tcc(n, d_out, rblk)
    gsum = tcc(S21[:n], h2p1, deg1, g1_b2.reshape(1, -1),
               S22[:n], h2p2, deg2, g2_b2.reshape(1, -1))

    gids = jnp.concatenate([head_ids, tail_ids]).astype(I32)
    gp_pad = jnp.concatenate(
        [gene_pathways.astype(I32),
         jnp.zeros((gene_pathways.shape[0], DW - ppg), I32)], axis=1)
    gene_k = _make_gene_kernel(n, ppg, d_out, roots, 2 * b)
    HT = gene_k(gsum, gids, gp_pad, pathway_root.astype(I32))
    HTf = HT.reshape(2 * b, roots * d_out)

    tail1 = _make_tail1(2 * b, roots * d_out, pl1_W.shape[1], gdim, 256)
    F = tail1(HTf, pl1_W, pl1_b.reshape(1, -1), bn1_g.reshape(1, -1),
              bn1_b.reshape(1, -1), pl2_W, pl2_b.reshape(1, -1),
              bn2_g.reshape(1, -1), bn2_b.reshape(1, -1))

    tail2 = _make_tail2(b, gdim, fc_W.shape[1], fc1_W.shape[1], 256)
    out = tail2(F, F, fc_W, fc_b.reshape(1, -1), fc1_W, fc1_b.reshape(1, -1),
                fc2_W.reshape(1, -1), fc2_b.reshape(1, 1))
    return out
